# baseline (speedup 1.0000x reference)
import functools

import jax
import jax.numpy as jnp
from jax import lax
from jax.experimental import pallas as pl
from jax.experimental.pallas import tpu as pltpu
from jax.experimental.pallas import tpu_sc as plsc

_NX, _NY, _NZ, _C, _B, _N = 400, 400, 4, 64, 2, 30000
_NW = 32
_SLICES = (_B * _NX) // _NW
_TILE_W = _SLICES * _NY
_CW = _NZ * _TILE_W
_PLANE = _B * _NX * _NY
_CH = 2000
_NCHUNK = _N // _CH
_VREGS = _CH // 16
_UNROLL = 5


def _proj_body(w1_ref, f_ref, ct_ref, ne_ref, b1_ref, g1_ref, be1_ref,
               w2_ref, b2_ref, g2_ref, be2_ref,
               p0_ref, p1_ref, p2_ref, p3_ref, gk_ref, prm_ref):
    p = lax.dot_general(
        w1_ref[...], f_ref[...], (((0,), (1,)), ((), ())),
        preferred_element_type=jnp.float32)
    p0_ref[...] = p[0, :]
    p1_ref[...] = p[1, :]
    p2_ref[...] = p[2, :]
    p3_ref[...] = p[3, :]
    b = jnp.minimum(ct_ref[0:1, :], _B - 1)
    z = jnp.minimum(ct_ref[1:2, :], _NZ - 1)
    x = jnp.minimum(ct_ref[2:3, :], _NX - 1)
    y = jnp.minimum(ct_ref[3:4, :], _NY - 1)
    bx = b * _NX + x
    gk_ref[...] = (z * _PLANE + bx * _NY + y)[0, :]
    cvec = jnp.sum(ne_ref[...]) * jnp.sum(w1_ref[...], axis=0) + b1_ref[...]
    prm_ref[0:1, :] = cvec.reshape(1, _NZ)
    prm_ref[1:2, :] = g1_ref[...].reshape(1, _NZ)
    prm_ref[2:3, :] = be1_ref[...].reshape(1, _NZ)
    prm_ref[3:4, :] = b2_ref[...].reshape(1, _NZ)
    prm_ref[4:5, :] = g2_ref[...].reshape(1, _NZ)
    prm_ref[5:6, :] = be2_ref[...].reshape(1, _NZ)
    prm_ref[6:10, :] = w2_ref[...]


def _project(voxel_features, W1, coorsT, nz_embed, b1, g1, be1, W2, b2, g2,
             be2):
    vec_f = jax.ShapeDtypeStruct((_N,), jnp.float32)
    vec_i = jax.ShapeDtypeStruct((_N,), jnp.int32)
    return pl.pallas_call(
        _proj_body,
        out_shape=[vec_f, vec_f, vec_f, vec_f, vec_i,
                   jax.ShapeDtypeStruct((10, _NZ), jnp.float32)],
    )(W1, voxel_features, coorsT, nz_embed, b1, g1, be1, W2, b2, g2, be2)


def _sc_body(gkh, p0h, p1h, p2h, p3h, zh, h1_hbm, bin_hbm,
             widx, h1acc, bgk0, bgk1, bp00, bp01, bp10, bp11,
             bp20, bp21, bp30, bp31, sem):
    bgk = [bgk0, bgk1]
    bp0 = [bp00, bp01]
    bp1 = [bp10, bp11]
    bp2 = [bp20, bp21]
    bp3 = [bp30, bp31]
    wid = lax.axis_index("s") * 2 + lax.axis_index("c")
    lo4 = wid * _TILE_W
    lane = jnp.arange(16, dtype=jnp.int32)

    pltpu.sync_copy(zh, widx)
    pltpu.sync_copy(zh, h1acc)

    def _keys(bgk_s, j):
        o = pl.multiple_of(j * 16, 8)
        g = bgk_s[pl.ds(o, 16)]
        g1m = jnp.where(g >= 2 * _PLANE, g - 2 * _PLANE, g)
        r = jnp.where(g1m >= _PLANE, g1m - _PLANE, g1m)
        msk = (r >= lo4) & (r < lo4 + _TILE_W)
        h = jnp.clip(r - lo4, 0, _TILE_W - 1)
        idx = h + ((g - r) >> 5)
        return o, idx, h, msk

    def _issue1(ci, s):
        base = pl.multiple_of(ci * _CH, 8)
        return [pltpu.async_copy(gkh.at[pl.ds(base, _CH)], bgk[s], sem)]

    hs = _issue1(0, 0)
    for ci in range(_NCHUNK):
        s = ci % 2
        for h in hs:
            h.wait()
        if ci + 1 < _NCHUNK:
            hs = _issue1(ci + 1, 1 - s)
        base = ci * _CH

        def _p1_vec(jj, c2, s=s, base=base):
            for u in range(_UNROLL):
                j = jj * _UNROLL + u
                o, idx, h, msk = _keys(bgk[s], j)
                pid = (lane + (base + j * 16 + 1)).astype(jnp.float32)
                plsc.store_scatter(widx, [idx], pid, mask=msk)
            return c2
        lax.fori_loop(0, _VREGS // _UNROLL, _p1_vec, 0)

    def _issue2(ci, s):
        base = pl.multiple_of(ci * _CH, 8)
        return [pltpu.async_copy(gkh.at[pl.ds(base, _CH)], bgk[s], sem),
                pltpu.async_copy(p0h.at[pl.ds(base, _CH)], bp0[s], sem),
                pltpu.async_copy(p1h.at[pl.ds(base, _CH)], bp1[s], sem),
                pltpu.async_copy(p2h.at[pl.ds(base, _CH)], bp2[s], sem),
                pltpu.async_copy(p3h.at[pl.ds(base, _CH)], bp3[s], sem)]

    hs = _issue2(0, 0)
    for ci in range(_NCHUNK):
        s = ci % 2
        for h in hs:
            h.wait()
        if ci + 1 < _NCHUNK:
            hs = _issue2(ci + 1, 1 - s)
        base = ci * _CH

        def _p2_vec(jj, c2, s=s, base=base):
            for u in range(_UNROLL):
                j = jj * _UNROLL + u
                o, idx, h, msk = _keys(bgk[s], j)
                pid = (lane + (base + j * 16 + 1)).astype(jnp.float32)
                w = plsc.load_gather(widx, [idx], mask=msk)
                keep = msk & (w == pid)
                plsc.addupdate_scatter(h1acc, [h], bp0[s][pl.ds(o, 16)],
                                       mask=keep)
                plsc.addupdate_scatter(h1acc, [h + _TILE_W],
                                       bp1[s][pl.ds(o, 16)], mask=keep)
                plsc.addupdate_scatter(h1acc, [h + 2 * _TILE_W],
                                       bp2[s][pl.ds(o, 16)], mask=keep)
                plsc.addupdate_scatter(h1acc, [h + 3 * _TILE_W],
                                       bp3[s][pl.ds(o, 16)], mask=keep)
            return c2
        lax.fori_loop(0, _VREGS // _UNROLL, _p2_vec, 0)

    def _conv(i, carry):
        for u in range(10):
            o = pl.multiple_of((i * 10 + u) * 16, 8)
            w = widx[pl.ds(o, 16)]
            widx[pl.ds(o, 16)] = jnp.where(w > 0.0, 1.0, 0.0)
        return carry
    lax.fori_loop(0, _CW // 160, _conv, 0)

    bb = wid // (_NX // _SLICES)
    xlo = wid * _SLICES - bb * _NX
    for z in range(_NZ):
        off = pl.multiple_of(((bb * _NZ + z) * _NX + xlo) * _NY, 8)
        pltpu.sync_copy(widx.at[pl.ds(z * _TILE_W, _TILE_W)],
                        bin_hbm.at[pl.ds(off, _TILE_W)])
    for c in range(_NZ):
        off = pl.multiple_of(c * (_B * _NX * _NY) + wid * _TILE_W, 8)
        pltpu.sync_copy(h1acc.at[pl.ds(c * _TILE_W, _TILE_W)],
                        h1_hbm.at[pl.ds(off, _TILE_W)])


def _sc_scatter(gk, p0, p1, p2, p3, zeros_cw):
    mesh = plsc.VectorSubcoreMesh(core_axis_name="c", subcore_axis_name="s")
    fn = functools.partial(
        pl.kernel, mesh=mesh,
        compiler_params=pltpu.CompilerParams(needs_layout_passes=False),
        out_type=[
            jax.ShapeDtypeStruct((_NZ * _B * _NX * _NY,), jnp.float32),
            jax.ShapeDtypeStruct((_B * _NZ * _NX * _NY,), jnp.float32),
        ],
        scratch_types=[
            pltpu.VMEM((_CW,), jnp.float32),
            pltpu.VMEM((_CW,), jnp.float32),
            pltpu.VMEM((_CH,), jnp.int32),
            pltpu.VMEM((_CH,), jnp.int32),
            pltpu.VMEM((_CH,), jnp.float32),
            pltpu.VMEM((_CH,), jnp.float32),
            pltpu.VMEM((_CH,), jnp.float32),
            pltpu.VMEM((_CH,), jnp.float32),
            pltpu.VMEM((_CH,), jnp.float32),
            pltpu.VMEM((_CH,), jnp.float32),
            pltpu.VMEM((_CH,), jnp.float32),
            pltpu.VMEM((_CH,), jnp.float32),
            pltpu.SemaphoreType.DMA,
        ],
    )(_sc_body)
    return fn(gk, p0, p1, p2, p3, zeros_cw)


def _head_body(p_ref, h1_ref, o_ref):
    a = [h1_ref[c, 0] + p_ref[0, c] for c in range(4)]
    m = (a[0] + a[1] + a[2] + a[3]) * 0.25
    d = [a[c] - m for c in range(4)]
    var = (d[0] * d[0] + d[1] * d[1] + d[2] * d[2] + d[3] * d[3]) * 0.25
    inv = lax.rsqrt(var + 1e-5)
    r = [jnp.maximum(d[c] * inv * p_ref[1, c] + p_ref[2, c], 0.0)
         for c in range(4)]
    s = [r[0] * p_ref[6, j] + r[1] * p_ref[7, j] + r[2] * p_ref[8, j]
         + r[3] * p_ref[9, j] + p_ref[3, j] for j in range(4)]
    m2 = (s[0] + s[1] + s[2] + s[3]) * 0.25
    d2 = [s[j] - m2 for j in range(4)]
    var2 = (d2[0] * d2[0] + d2[1] * d2[1] + d2[2] * d2[2]
            + d2[3] * d2[3]) * 0.25
    inv2 = lax.rsqrt(var2 + 1e-5)
    for j in range(4):
        o_ref[0, j] = d2[j] * inv2 * p_ref[4, j] + p_ref[5, j]


def _head(P, H1):
    bx = 40
    return pl.pallas_call(
        _head_body,
        grid=(_B, _NX // bx),
        in_specs=[
            pl.BlockSpec(memory_space=pltpu.SMEM),
            pl.BlockSpec((_NZ, 1, bx, _NY), lambda b, i: (0, b, i, 0)),
        ],
        out_specs=pl.BlockSpec((1, _NZ, bx, _NY), lambda b, i: (b, 0, i, 0)),
        out_shape=jax.ShapeDtypeStruct((_B, _NZ, _NX, _NY), jnp.float32),
    )(P, H1)


def kernel(voxel_features, coors, nz_embed, W1, b1, g1, be1, W2, b2, g2, be2,
           batch_size):
    p0, p1, p2, p3, gk, P = _project(
        voxel_features, W1, coors.T, nz_embed, b1, g1, be1, W2, b2, g2, be2)
    h1_flat, bin_flat = _sc_scatter(
        gk, p0, p1, p2, p3, jnp.zeros((_CW,), jnp.float32))
    H1 = h1_flat.reshape(_NZ, _B, _NX, _NY)
    out = _head(P, H1)
    binary = bin_flat.reshape(_B, _NZ, _NX, _NY)
    return out, binary


# speedup vs baseline: 1.0609x; 1.0609x over previous
import functools

import jax
import jax.numpy as jnp
from jax import lax
from jax.experimental import pallas as pl
from jax.experimental.pallas import tpu as pltpu
from jax.experimental.pallas import tpu_sc as plsc

_NX, _NY, _NZ, _C, _B, _N = 400, 400, 4, 64, 2, 30000
_NW = 32
_SLICES = (_B * _NX) // _NW
_TILE_W = _SLICES * _NY
_CW = _NZ * _TILE_W
_PLANE = _B * _NX * _NY
_CH = 2000
_NCHUNK = _N // _CH
_VREGS = _CH // 16
_UNROLL = 5


def _proj_body(w1_ref, f_ref, ct_ref, ne_ref, b1_ref, g1_ref, be1_ref,
               w2_ref, b2_ref, g2_ref, be2_ref,
               p0_ref, p1_ref, p2_ref, p3_ref, gw_ref, hw_ref, prm_ref):
    p = lax.dot_general(
        w1_ref[...], f_ref[...], (((0,), (1,)), ((), ())),
        preferred_element_type=jnp.float32)
    p0_ref[...] = p[0, :]
    p1_ref[...] = p[1, :]
    p2_ref[...] = p[2, :]
    p3_ref[...] = p[3, :]
    b = jnp.minimum(ct_ref[0:1, :], _B - 1)
    z = jnp.minimum(ct_ref[1:2, :], _NZ - 1)
    x = jnp.minimum(ct_ref[2:3, :], _NX - 1)
    y = jnp.minimum(ct_ref[3:4, :], _NY - 1)
    hw = b * _NX * _NY + x * _NY + y
    hw_ref[...] = hw[0, :]
    gw_ref[...] = (z * _TILE_W + hw)[0, :]
    cvec = jnp.sum(ne_ref[...]) * jnp.sum(w1_ref[...], axis=0) + b1_ref[...]
    prm_ref[0:1, :] = cvec.reshape(1, _NZ)
    prm_ref[1:2, :] = g1_ref[...].reshape(1, _NZ)
    prm_ref[2:3, :] = be1_ref[...].reshape(1, _NZ)
    prm_ref[3:4, :] = b2_ref[...].reshape(1, _NZ)
    prm_ref[4:5, :] = g2_ref[...].reshape(1, _NZ)
    prm_ref[5:6, :] = be2_ref[...].reshape(1, _NZ)
    prm_ref[6:10, :] = w2_ref[...]


def _project(voxel_features, W1, coorsT, nz_embed, b1, g1, be1, W2, b2, g2,
             be2):
    vec_f = jax.ShapeDtypeStruct((_N,), jnp.float32)
    vec_i = jax.ShapeDtypeStruct((_N,), jnp.int32)
    return pl.pallas_call(
        _proj_body,
        out_shape=[vec_f, vec_f, vec_f, vec_f, vec_i, vec_i,
                   jax.ShapeDtypeStruct((10, _NZ), jnp.float32)],
    )(W1, voxel_features, coorsT, nz_embed, b1, g1, be1, W2, b2, g2, be2)


def _sc_body(gwh, hwh, p0h, p1h, p2h, p3h, zh, h1_hbm, bin_hbm,
             widx, h1acc, bgw0, bgw1, bhw0, bhw1, bp00, bp01, bp10, bp11,
             bp20, bp21, bp30, bp31, sem):
    bgw = [bgw0, bgw1]
    bhw = [bhw0, bhw1]
    bp0 = [bp00, bp01]
    bp1 = [bp10, bp11]
    bp2 = [bp20, bp21]
    bp3 = [bp30, bp31]
    wid = lax.axis_index("s") * 2 + lax.axis_index("c")
    lo4 = wid * _TILE_W
    lanef = jnp.arange(16, dtype=jnp.int32).astype(jnp.float32)

    pltpu.sync_copy(zh, widx)
    pltpu.sync_copy(zh, h1acc)

    def _keys(s, j):
        o = pl.multiple_of(j * 16, 8)
        hw = bhw[s][pl.ds(o, 16)]
        msk = (hw >= lo4) & (hw < lo4 + _TILE_W)
        h = jnp.clip(hw - lo4, 0, _TILE_W - 1)
        idx = jnp.clip(bgw[s][pl.ds(o, 16)] - lo4, 0, _CW - 1)
        return o, idx, h, msk

    def _issue1(ci, s):
        base = pl.multiple_of(ci * _CH, 8)
        return [pltpu.async_copy(gwh.at[pl.ds(base, _CH)], bgw[s], sem),
                pltpu.async_copy(hwh.at[pl.ds(base, _CH)], bhw[s], sem)]

    hs = _issue1(0, 0)
    for ci in range(_NCHUNK):
        s = ci % 2
        for h in hs:
            h.wait()
        if ci + 1 < _NCHUNK:
            hs = _issue1(ci + 1, 1 - s)
        base = ci * _CH

        def _p1_vec(jj, c2, s=s, base=base):
            bf = (jj * (16 * _UNROLL)).astype(jnp.float32)
            for u in range(_UNROLL):
                j = jj * _UNROLL + u
                o, idx, h, msk = _keys(s, j)
                pid = lanef + (bf + (base + u * 16 + 1))
                plsc.store_scatter(widx, [idx], pid, mask=msk)
            return c2
        lax.fori_loop(0, _VREGS // _UNROLL, _p1_vec, 0)

    def _issue2(ci, s):
        base = pl.multiple_of(ci * _CH, 8)
        return [pltpu.async_copy(gwh.at[pl.ds(base, _CH)], bgw[s], sem),
                pltpu.async_copy(hwh.at[pl.ds(base, _CH)], bhw[s], sem),
                pltpu.async_copy(p0h.at[pl.ds(base, _CH)], bp0[s], sem),
                pltpu.async_copy(p1h.at[pl.ds(base, _CH)], bp1[s], sem),
                pltpu.async_copy(p2h.at[pl.ds(base, _CH)], bp2[s], sem),
                pltpu.async_copy(p3h.at[pl.ds(base, _CH)], bp3[s], sem)]

    hs = _issue2(0, 0)
    for ci in range(_NCHUNK):
        s = ci % 2
        for h in hs:
            h.wait()
        if ci + 1 < _NCHUNK:
            hs = _issue2(ci + 1, 1 - s)
        base = ci * _CH

        def _p2_vec(jj, c2, s=s, base=base):
            bf = (jj * (16 * _UNROLL)).astype(jnp.float32)
            for u in range(_UNROLL):
                j = jj * _UNROLL + u
                o, idx, h, msk = _keys(s, j)
                pid = lanef + (bf + (base + u * 16 + 1))
                w = plsc.load_gather(widx, [idx], mask=msk)
                keep = msk & (w == pid)
                plsc.addupdate_scatter(h1acc, [h], bp0[s][pl.ds(o, 16)],
                                       mask=keep)
                plsc.addupdate_scatter(h1acc, [h + _TILE_W],
                                       bp1[s][pl.ds(o, 16)], mask=keep)
                plsc.addupdate_scatter(h1acc, [h + 2 * _TILE_W],
                                       bp2[s][pl.ds(o, 16)], mask=keep)
                plsc.addupdate_scatter(h1acc, [h + 3 * _TILE_W],
                                       bp3[s][pl.ds(o, 16)], mask=keep)
            return c2
        lax.fori_loop(0, _VREGS // _UNROLL, _p2_vec, 0)

    def _conv(i, carry):
        for u in range(10):
            o = pl.multiple_of((i * 10 + u) * 16, 8)
            w = widx[pl.ds(o, 16)]
            widx[pl.ds(o, 16)] = jnp.where(w > 0.0, 1.0, 0.0)
        return carry
    lax.fori_loop(0, _CW // 160, _conv, 0)

    bb = wid // (_NX // _SLICES)
    xlo = wid * _SLICES - bb * _NX
    for z in range(_NZ):
        off = pl.multiple_of(((bb * _NZ + z) * _NX + xlo) * _NY, 8)
        pltpu.sync_copy(widx.at[pl.ds(z * _TILE_W, _TILE_W)],
                        bin_hbm.at[pl.ds(off, _TILE_W)])
    for c in range(_NZ):
        off = pl.multiple_of(c * (_B * _NX * _NY) + wid * _TILE_W, 8)
        pltpu.sync_copy(h1acc.at[pl.ds(c * _TILE_W, _TILE_W)],
                        h1_hbm.at[pl.ds(off, _TILE_W)])


def _sc_scatter(gw, hw, p0, p1, p2, p3, zeros_cw):
    mesh = plsc.VectorSubcoreMesh(core_axis_name="c", subcore_axis_name="s")
    fn = functools.partial(
        pl.kernel, mesh=mesh,
        compiler_params=pltpu.CompilerParams(needs_layout_passes=False),
        out_type=[
            jax.ShapeDtypeStruct((_NZ * _B * _NX * _NY,), jnp.float32),
            jax.ShapeDtypeStruct((_B * _NZ * _NX * _NY,), jnp.float32),
        ],
        scratch_types=[
            pltpu.VMEM((_CW,), jnp.float32),
            pltpu.VMEM((_CW,), jnp.float32),
            pltpu.VMEM((_CH,), jnp.int32),
            pltpu.VMEM((_CH,), jnp.int32),
            pltpu.VMEM((_CH,), jnp.int32),
            pltpu.VMEM((_CH,), jnp.int32),
            pltpu.VMEM((_CH,), jnp.float32),
            pltpu.VMEM((_CH,), jnp.float32),
            pltpu.VMEM((_CH,), jnp.float32),
            pltpu.VMEM((_CH,), jnp.float32),
            pltpu.VMEM((_CH,), jnp.float32),
            pltpu.VMEM((_CH,), jnp.float32),
            pltpu.VMEM((_CH,), jnp.float32),
            pltpu.VMEM((_CH,), jnp.float32),
            pltpu.SemaphoreType.DMA,
        ],
    )(_sc_body)
    return fn(gw, hw, p0, p1, p2, p3, zeros_cw)


def _head_body(p_ref, h1_ref, o_ref):
    a = [h1_ref[c, 0] + p_ref[0, c] for c in range(4)]
    m = (a[0] + a[1] + a[2] + a[3]) * 0.25
    d = [a[c] - m for c in range(4)]
    var = (d[0] * d[0] + d[1] * d[1] + d[2] * d[2] + d[3] * d[3]) * 0.25
    inv = lax.rsqrt(var + 1e-5)
    r = [jnp.maximum(d[c] * inv * p_ref[1, c] + p_ref[2, c], 0.0)
         for c in range(4)]
    s = [r[0] * p_ref[6, j] + r[1] * p_ref[7, j] + r[2] * p_ref[8, j]
         + r[3] * p_ref[9, j] + p_ref[3, j] for j in range(4)]
    m2 = (s[0] + s[1] + s[2] + s[3]) * 0.25
    d2 = [s[j] - m2 for j in range(4)]
    var2 = (d2[0] * d2[0] + d2[1] * d2[1] + d2[2] * d2[2]
            + d2[3] * d2[3]) * 0.25
    inv2 = lax.rsqrt(var2 + 1e-5)
    for j in range(4):
        o_ref[0, j] = d2[j] * inv2 * p_ref[4, j] + p_ref[5, j]


def _head(P, H1):
    bx = 40
    return pl.pallas_call(
        _head_body,
        grid=(_B, _NX // bx),
        in_specs=[
            pl.BlockSpec(memory_space=pltpu.SMEM),
            pl.BlockSpec((_NZ, 1, bx, _NY), lambda b, i: (0, b, i, 0)),
        ],
        out_specs=pl.BlockSpec((1, _NZ, bx, _NY), lambda b, i: (b, 0, i, 0)),
        out_shape=jax.ShapeDtypeStruct((_B, _NZ, _NX, _NY), jnp.float32),
    )(P, H1)


def kernel(voxel_features, coors, nz_embed, W1, b1, g1, be1, W2, b2, g2, be2,
           batch_size):
    p0, p1, p2, p3, gw, hw, P = _project(
        voxel_features, W1, coors.T, nz_embed, b1, g1, be1, W2, b2, g2, be2)
    h1_flat, bin_flat = _sc_scatter(
        gw, hw, p0, p1, p2, p3, jnp.zeros((_CW,), jnp.float32))
    H1 = h1_flat.reshape(_NZ, _B, _NX, _NY)
    out = _head(P, H1)
    binary = bin_flat.reshape(_B, _NZ, _NX, _NY)
    return out, binary


# speedup vs baseline: 1.2952x; 1.2208x over previous
import functools

import jax
import jax.numpy as jnp
from jax import lax
from jax.experimental import pallas as pl
from jax.experimental.pallas import tpu as pltpu
from jax.experimental.pallas import tpu_sc as plsc

_NX, _NY, _NZ, _C, _B, _N = 400, 400, 4, 64, 2, 30000
_NW = 32
_SLICES = (_B * _NX) // _NW
_TILE_W = _SLICES * _NY
_CW = _NZ * _TILE_W
_PLANE = _B * _NX * _NY
_CH = 2000
_NCHUNK = _N // _CH
_VREGS = _CH // 16
_UNROLL = 5


def _proj_body(w1_ref, f_ref, ct_ref, ne_ref, b1_ref, g1_ref, be1_ref,
               w2_ref, b2_ref, g2_ref, be2_ref,
               p0_ref, p1_ref, p2_ref, p3_ref, gw_ref, hw_ref, prm_ref):
    p = lax.dot_general(
        w1_ref[...], f_ref[...], (((0,), (1,)), ((), ())),
        preferred_element_type=jnp.float32)
    p0_ref[...] = p[0, :]
    p1_ref[...] = p[1, :]
    p2_ref[...] = p[2, :]
    p3_ref[...] = p[3, :]
    b = jnp.minimum(ct_ref[0:1, :], _B - 1)
    z = jnp.minimum(ct_ref[1:2, :], _NZ - 1)
    x = jnp.minimum(ct_ref[2:3, :], _NX - 1)
    y = jnp.minimum(ct_ref[3:4, :], _NY - 1)
    hw = b * _NX * _NY + x * _NY + y
    hw_ref[...] = hw[0, :]
    gw_ref[...] = (z * _TILE_W + hw)[0, :]
    cvec = jnp.sum(ne_ref[...]) * jnp.sum(w1_ref[...], axis=0) + b1_ref[...]
    prm_ref[0:1, :] = cvec.reshape(1, _NZ)
    prm_ref[1:2, :] = g1_ref[...].reshape(1, _NZ)
    prm_ref[2:3, :] = be1_ref[...].reshape(1, _NZ)
    prm_ref[3:4, :] = b2_ref[...].reshape(1, _NZ)
    prm_ref[4:5, :] = g2_ref[...].reshape(1, _NZ)
    prm_ref[5:6, :] = be2_ref[...].reshape(1, _NZ)
    prm_ref[6:10, :] = w2_ref[...]


def _project(voxel_features, W1, coorsT, nz_embed, b1, g1, be1, W2, b2, g2,
             be2):
    vec_f = jax.ShapeDtypeStruct((_N,), jnp.float32)
    vec_i = jax.ShapeDtypeStruct((_N,), jnp.int32)
    return pl.pallas_call(
        _proj_body,
        out_shape=[vec_f, vec_f, vec_f, vec_f, vec_i, vec_i,
                   jax.ShapeDtypeStruct((10, _NZ), jnp.float32)],
    )(W1, voxel_features, coorsT, nz_embed, b1, g1, be1, W2, b2, g2, be2)


def _sc_body(gwh, hwh, p0h, p1h, p2h, p3h, zh, h1_hbm, bin_hbm,
             widx, h1acc, bgw0, bgw1, bhw0, bhw1, bp00, bp01, bp10, bp11,
             bp20, bp21, bp30, bp31, sem0, sem1):
    bgw = [bgw0, bgw1]
    bhw = [bhw0, bhw1]
    bp0 = [bp00, bp01]
    bp1 = [bp10, bp11]
    bp2 = [bp20, bp21]
    bp3 = [bp30, bp31]
    wid = lax.axis_index("s") * 2 + lax.axis_index("c")
    lo4 = wid * _TILE_W
    lanef = jnp.arange(16, dtype=jnp.int32).astype(jnp.float32)

    pltpu.sync_copy(zh, widx)
    pltpu.sync_copy(zh, h1acc)

    def _keys(s, j):
        o = pl.multiple_of(j * 16, 8)
        hw = bhw[s][pl.ds(o, 16)]
        msk = (hw >= lo4) & (hw < lo4 + _TILE_W)
        h = jnp.clip(hw - lo4, 0, _TILE_W - 1)
        idx = jnp.clip(bgw[s][pl.ds(o, 16)] - lo4, 0, _CW - 1)
        return o, idx, h, msk

    sems = [sem0, sem1]

    def _issue1(base, s):
        base = pl.multiple_of(base, 8)
        pltpu.async_copy(gwh.at[pl.ds(base, _CH)], bgw[s], sems[s])
        pltpu.async_copy(hwh.at[pl.ds(base, _CH)], bhw[s], sems[s])

    def _wait1(s):
        pltpu.make_async_copy(gwh.at[pl.ds(0, _CH)], bgw[s], sems[s]).wait()
        pltpu.make_async_copy(hwh.at[pl.ds(0, _CH)], bhw[s], sems[s]).wait()

    def _p1_compute(s, base):
        def _p1_vec(jj, c2):
            for u in range(_UNROLL):
                j = jj * _UNROLL + u
                o, idx, h, msk = _keys(s, j)
                pid = lanef + (base + j * 16 + 1).astype(jnp.float32)
                plsc.store_scatter(widx, [idx], pid, mask=msk)
            return c2
        lax.fori_loop(0, _VREGS // _UNROLL, _p1_vec, 0)

    _issue1(0, 0)

    def _p1_pair(ci2, carry):
        base0 = ci2 * (2 * _CH)
        _wait1(0)
        _issue1(base0 + _CH, 1)
        _p1_compute(0, base0)
        _wait1(1)
        _issue1(base0 + 2 * _CH, 0)
        _p1_compute(1, base0 + _CH)
        return carry
    lax.fori_loop(0, (_NCHUNK - 1) // 2, _p1_pair, 0)
    _wait1(0)
    _p1_compute(0, (_NCHUNK - 1) * _CH)

    def _issue2(base, s):
        base = pl.multiple_of(base, 8)
        pltpu.async_copy(gwh.at[pl.ds(base, _CH)], bgw[s], sems[s])
        pltpu.async_copy(hwh.at[pl.ds(base, _CH)], bhw[s], sems[s])
        pltpu.async_copy(p0h.at[pl.ds(base, _CH)], bp0[s], sems[s])
        pltpu.async_copy(p1h.at[pl.ds(base, _CH)], bp1[s], sems[s])
        pltpu.async_copy(p2h.at[pl.ds(base, _CH)], bp2[s], sems[s])
        pltpu.async_copy(p3h.at[pl.ds(base, _CH)], bp3[s], sems[s])

    def _wait2(s):
        _wait1(s)
        pltpu.make_async_copy(p0h.at[pl.ds(0, _CH)], bp0[s], sems[s]).wait()
        pltpu.make_async_copy(p1h.at[pl.ds(0, _CH)], bp1[s], sems[s]).wait()
        pltpu.make_async_copy(p2h.at[pl.ds(0, _CH)], bp2[s], sems[s]).wait()
        pltpu.make_async_copy(p3h.at[pl.ds(0, _CH)], bp3[s], sems[s]).wait()

    def _p2_compute(s, base):
        @plsc.parallel_loop(0, _VREGS // _UNROLL, 1)
        def _p2_vec(jj):
            for u in range(_UNROLL):
                j = jj * _UNROLL + u
                o, idx, h, msk = _keys(s, j)
                pid = lanef + (base + j * 16 + 1).astype(jnp.float32)
                w = plsc.load_gather(widx, [idx], mask=msk)
                keep = msk & (w == pid)
                plsc.addupdate_scatter(h1acc, [h], bp0[s][pl.ds(o, 16)],
                                       mask=keep)
                plsc.addupdate_scatter(h1acc, [h + _TILE_W],
                                       bp1[s][pl.ds(o, 16)], mask=keep)
                plsc.addupdate_scatter(h1acc, [h + 2 * _TILE_W],
                                       bp2[s][pl.ds(o, 16)], mask=keep)
                plsc.addupdate_scatter(h1acc, [h + 3 * _TILE_W],
                                       bp3[s][pl.ds(o, 16)], mask=keep)

    _issue2(0, 0)

    def _p2_pair(ci2, carry):
        base0 = ci2 * (2 * _CH)
        _wait2(0)
        _issue2(base0 + _CH, 1)
        _p2_compute(0, base0)
        _wait2(1)
        _issue2(base0 + 2 * _CH, 0)
        _p2_compute(1, base0 + _CH)
        return carry
    lax.fori_loop(0, (_NCHUNK - 1) // 2, _p2_pair, 0)
    _wait2(0)
    _p2_compute(0, (_NCHUNK - 1) * _CH)

    @plsc.parallel_loop(0, _CW // 160, 1)
    def _conv(i):
        for u in range(10):
            o = pl.multiple_of((i * 10 + u) * 16, 8)
            w = widx[pl.ds(o, 16)]
            widx[pl.ds(o, 16)] = jnp.where(w > 0.0, 1.0, 0.0)

    bb = wid // (_NX // _SLICES)
    xlo = wid * _SLICES - bb * _NX
    for z in range(_NZ):
        off = pl.multiple_of(((bb * _NZ + z) * _NX + xlo) * _NY, 8)
        pltpu.sync_copy(widx.at[pl.ds(z * _TILE_W, _TILE_W)],
                        bin_hbm.at[pl.ds(off, _TILE_W)])
    for c in range(_NZ):
        off = pl.multiple_of(c * (_B * _NX * _NY) + wid * _TILE_W, 8)
        pltpu.sync_copy(h1acc.at[pl.ds(c * _TILE_W, _TILE_W)],
                        h1_hbm.at[pl.ds(off, _TILE_W)])


def _sc_scatter(gw, hw, p0, p1, p2, p3, zeros_cw):
    mesh = plsc.VectorSubcoreMesh(core_axis_name="c", subcore_axis_name="s")
    fn = functools.partial(
        pl.kernel, mesh=mesh,
        compiler_params=pltpu.CompilerParams(needs_layout_passes=False),
        out_type=[
            jax.ShapeDtypeStruct((_NZ * _B * _NX * _NY,), jnp.float32),
            jax.ShapeDtypeStruct((_B * _NZ * _NX * _NY,), jnp.float32),
        ],
        scratch_types=[
            pltpu.VMEM((_CW,), jnp.float32),
            pltpu.VMEM((_CW,), jnp.float32),
            pltpu.VMEM((_CH,), jnp.int32),
            pltpu.VMEM((_CH,), jnp.int32),
            pltpu.VMEM((_CH,), jnp.int32),
            pltpu.VMEM((_CH,), jnp.int32),
            pltpu.VMEM((_CH,), jnp.float32),
            pltpu.VMEM((_CH,), jnp.float32),
            pltpu.VMEM((_CH,), jnp.float32),
            pltpu.VMEM((_CH,), jnp.float32),
            pltpu.VMEM((_CH,), jnp.float32),
            pltpu.VMEM((_CH,), jnp.float32),
            pltpu.VMEM((_CH,), jnp.float32),
            pltpu.VMEM((_CH,), jnp.float32),
            pltpu.SemaphoreType.DMA,
            pltpu.SemaphoreType.DMA,
        ],
    )(_sc_body)
    return fn(gw, hw, p0, p1, p2, p3, zeros_cw)


def _head_body(p_ref, h1_ref, o_ref):
    a = [h1_ref[c, 0] + p_ref[0, c] for c in range(4)]
    m = (a[0] + a[1] + a[2] + a[3]) * 0.25
    d = [a[c] - m for c in range(4)]
    var = (d[0] * d[0] + d[1] * d[1] + d[2] * d[2] + d[3] * d[3]) * 0.25
    inv = lax.rsqrt(var + 1e-5)
    r = [jnp.maximum(d[c] * inv * p_ref[1, c] + p_ref[2, c], 0.0)
         for c in range(4)]
    s = [r[0] * p_ref[6, j] + r[1] * p_ref[7, j] + r[2] * p_ref[8, j]
         + r[3] * p_ref[9, j] + p_ref[3, j] for j in range(4)]
    m2 = (s[0] + s[1] + s[2] + s[3]) * 0.25
    d2 = [s[j] - m2 for j in range(4)]
    var2 = (d2[0] * d2[0] + d2[1] * d2[1] + d2[2] * d2[2]
            + d2[3] * d2[3]) * 0.25
    inv2 = lax.rsqrt(var2 + 1e-5)
    for j in range(4):
        o_ref[0, j] = d2[j] * inv2 * p_ref[4, j] + p_ref[5, j]


def _head(P, H1):
    bx = 40
    return pl.pallas_call(
        _head_body,
        grid=(_B, _NX // bx),
        in_specs=[
            pl.BlockSpec(memory_space=pltpu.SMEM),
            pl.BlockSpec((_NZ, 1, bx, _NY), lambda b, i: (0, b, i, 0)),
        ],
        out_specs=pl.BlockSpec((1, _NZ, bx, _NY), lambda b, i: (b, 0, i, 0)),
        out_shape=jax.ShapeDtypeStruct((_B, _NZ, _NX, _NY), jnp.float32),
    )(P, H1)


def kernel(voxel_features, coors, nz_embed, W1, b1, g1, be1, W2, b2, g2, be2,
           batch_size):
    p0, p1, p2, p3, gw, hw, P = _project(
        voxel_features, W1, coors.T, nz_embed, b1, g1, be1, W2, b2, g2, be2)
    h1_flat, bin_flat = _sc_scatter(
        gw, hw, p0, p1, p2, p3, jnp.zeros((_CW,), jnp.float32))
    H1 = h1_flat.reshape(_NZ, _B, _NX, _NY)
    out = _head(P, H1)
    binary = bin_flat.reshape(_B, _NZ, _NX, _NY)
    return out, binary


# speedup vs baseline: 1.3791x; 1.0648x over previous
import functools

import jax
import jax.numpy as jnp
from jax import lax
from jax.experimental import pallas as pl
from jax.experimental.pallas import tpu as pltpu
from jax.experimental.pallas import tpu_sc as plsc

_NX, _NY, _NZ, _C, _B, _N = 400, 400, 4, 64, 2, 30000
_NW = 32
_SLICES = (_B * _NX) // _NW
_TILE_W = _SLICES * _NY
_CW = _NZ * _TILE_W
_PLANE = _B * _NX * _NY
_CH = 2000
_NCHUNK = _N // _CH
_VREGS = _CH // 16
_UNROLL = 5


def _proj_body(w1_ref, f_ref, ct_ref, ne_ref, b1_ref, g1_ref, be1_ref,
               w2_ref, b2_ref, g2_ref, be2_ref,
               p0_ref, p1_ref, p2_ref, p3_ref, gw_ref, hw_ref, prm_ref):
    p = lax.dot_general(
        w1_ref[...], f_ref[...], (((0,), (1,)), ((), ())),
        preferred_element_type=jnp.float32)
    p0_ref[...] = p[0, :]
    p1_ref[...] = p[1, :]
    p2_ref[...] = p[2, :]
    p3_ref[...] = p[3, :]
    b = jnp.minimum(ct_ref[0:1, :], _B - 1)
    z = jnp.minimum(ct_ref[1:2, :], _NZ - 1)
    x = jnp.minimum(ct_ref[2:3, :], _NX - 1)
    y = jnp.minimum(ct_ref[3:4, :], _NY - 1)
    hw = b * _NX * _NY + x * _NY + y
    hw_ref[...] = hw[0, :]
    gw_ref[...] = (z * _TILE_W + hw)[0, :]
    cvec = jnp.sum(ne_ref[...]) * jnp.sum(w1_ref[...], axis=0) + b1_ref[...]
    prm_ref[0:1, :] = cvec.reshape(1, _NZ)
    prm_ref[1:2, :] = g1_ref[...].reshape(1, _NZ)
    prm_ref[2:3, :] = be1_ref[...].reshape(1, _NZ)
    prm_ref[3:4, :] = b2_ref[...].reshape(1, _NZ)
    prm_ref[4:5, :] = g2_ref[...].reshape(1, _NZ)
    prm_ref[5:6, :] = be2_ref[...].reshape(1, _NZ)
    prm_ref[6:10, :] = w2_ref[...]


def _project(voxel_features, W1, coorsT, nz_embed, b1, g1, be1, W2, b2, g2,
             be2):
    vec_f = jax.ShapeDtypeStruct((_N,), jnp.float32)
    vec_i = jax.ShapeDtypeStruct((_N,), jnp.int32)
    return pl.pallas_call(
        _proj_body,
        out_shape=[vec_f, vec_f, vec_f, vec_f, vec_i, vec_i,
                   jax.ShapeDtypeStruct((10, _NZ), jnp.float32)],
    )(W1, voxel_features, coorsT, nz_embed, b1, g1, be1, W2, b2, g2, be2)


def _sc_body(gwh, hwh, p0h, p1h, p2h, p3h, h1_hbm, bin_hbm,
             widx, h1acc, bgw0, bgw1, bhw0, bhw1, bp00, bp01, bp10, bp11,
             bp20, bp21, bp30, bp31, sem0, sem1):
    bgw = [bgw0, bgw1]
    bhw = [bhw0, bhw1]
    bp0 = [bp00, bp01]
    bp1 = [bp10, bp11]
    bp2 = [bp20, bp21]
    bp3 = [bp30, bp31]
    wid = lax.axis_index("s") * 2 + lax.axis_index("c")
    lo4 = wid * _TILE_W
    lanef = jnp.arange(16, dtype=jnp.int32).astype(jnp.float32)

    @plsc.parallel_loop(0, _CW // 160, 1)
    def _zero(i):
        z16 = jnp.zeros((16,), jnp.float32)
        for u in range(10):
            o = pl.multiple_of((i * 10 + u) * 16, 8)
            widx[pl.ds(o, 16)] = z16
            h1acc[pl.ds(o, 16)] = z16

    def _keys(s, j):
        o = pl.multiple_of(j * 16, 8)
        hw = bhw[s][pl.ds(o, 16)]
        msk = (hw >= lo4) & (hw < lo4 + _TILE_W)
        h = jnp.clip(hw - lo4, 0, _TILE_W - 1)
        idx = jnp.clip(bgw[s][pl.ds(o, 16)] - lo4, 0, _CW - 1)
        return o, idx, h, msk

    sems = [sem0, sem1]

    def _issue1(base, s):
        base = pl.multiple_of(base, 8)
        pltpu.async_copy(gwh.at[pl.ds(base, _CH)], bgw[s], sems[s])
        pltpu.async_copy(hwh.at[pl.ds(base, _CH)], bhw[s], sems[s])

    def _wait1(s):
        pltpu.make_async_copy(gwh.at[pl.ds(0, _CH)], bgw[s], sems[s]).wait()
        pltpu.make_async_copy(hwh.at[pl.ds(0, _CH)], bhw[s], sems[s]).wait()

    def _p1_compute(s, base):
        def _p1_vec(jj, c2):
            for u in range(_UNROLL):
                j = jj * _UNROLL + u
                o, idx, h, msk = _keys(s, j)
                pid = lanef + (base + j * 16 + 1).astype(jnp.float32)
                plsc.store_scatter(widx, [idx], pid, mask=msk)
            return c2
        lax.fori_loop(0, _VREGS // _UNROLL, _p1_vec, 0)

    _issue1(0, 0)

    def _p1_pair(ci2, carry):
        base0 = ci2 * (2 * _CH)
        _wait1(0)
        _issue1(base0 + _CH, 1)
        _p1_compute(0, base0)
        _wait1(1)
        _issue1(base0 + 2 * _CH, 0)
        _p1_compute(1, base0 + _CH)
        return carry
    lax.fori_loop(0, (_NCHUNK - 1) // 2, _p1_pair, 0)
    _wait1(0)
    _p1_compute(0, (_NCHUNK - 1) * _CH)

    def _issue2(base, s):
        base = pl.multiple_of(base, 8)
        pltpu.async_copy(gwh.at[pl.ds(base, _CH)], bgw[s], sems[s])
        pltpu.async_copy(hwh.at[pl.ds(base, _CH)], bhw[s], sems[s])
        pltpu.async_copy(p0h.at[pl.ds(base, _CH)], bp0[s], sems[s])
        pltpu.async_copy(p1h.at[pl.ds(base, _CH)], bp1[s], sems[s])
        pltpu.async_copy(p2h.at[pl.ds(base, _CH)], bp2[s], sems[s])
        pltpu.async_copy(p3h.at[pl.ds(base, _CH)], bp3[s], sems[s])

    def _wait2(s):
        _wait1(s)
        pltpu.make_async_copy(p0h.at[pl.ds(0, _CH)], bp0[s], sems[s]).wait()
        pltpu.make_async_copy(p1h.at[pl.ds(0, _CH)], bp1[s], sems[s]).wait()
        pltpu.make_async_copy(p2h.at[pl.ds(0, _CH)], bp2[s], sems[s]).wait()
        pltpu.make_async_copy(p3h.at[pl.ds(0, _CH)], bp3[s], sems[s]).wait()

    def _p2_compute(s, base):
        @plsc.parallel_loop(0, _VREGS // _UNROLL, 1)
        def _p2_vec(jj):
            for u in range(_UNROLL):
                j = jj * _UNROLL + u
                o, idx, h, msk = _keys(s, j)
                pid = lanef + (base + j * 16 + 1).astype(jnp.float32)
                w = plsc.load_gather(widx, [idx], mask=msk)
                keep = msk & (w == pid)
                plsc.addupdate_scatter(h1acc, [h], bp0[s][pl.ds(o, 16)],
                                       mask=keep)
                plsc.addupdate_scatter(h1acc, [h + _TILE_W],
                                       bp1[s][pl.ds(o, 16)], mask=keep)
                plsc.addupdate_scatter(h1acc, [h + 2 * _TILE_W],
                                       bp2[s][pl.ds(o, 16)], mask=keep)
                plsc.addupdate_scatter(h1acc, [h + 3 * _TILE_W],
                                       bp3[s][pl.ds(o, 16)], mask=keep)

    _issue2(0, 0)

    def _p2_pair(ci2, carry):
        base0 = ci2 * (2 * _CH)
        _wait2(0)
        _issue2(base0 + _CH, 1)
        _p2_compute(0, base0)
        _wait2(1)
        _issue2(base0 + 2 * _CH, 0)
        _p2_compute(1, base0 + _CH)
        return carry
    lax.fori_loop(0, (_NCHUNK - 1) // 2, _p2_pair, 0)
    _wait2(0)
    _p2_compute(0, (_NCHUNK - 1) * _CH)

    for c in range(_NZ):
        off = pl.multiple_of(c * (_B * _NX * _NY) + wid * _TILE_W, 8)
        pltpu.async_copy(h1acc.at[pl.ds(c * _TILE_W, _TILE_W)],
                         h1_hbm.at[pl.ds(off, _TILE_W)], sem0)

    @plsc.parallel_loop(0, _CW // 160, 1)
    def _conv(i):
        for u in range(10):
            o = pl.multiple_of((i * 10 + u) * 16, 8)
            w = widx[pl.ds(o, 16)]
            widx[pl.ds(o, 16)] = jnp.where(w > 0.0, 1.0, 0.0)

    bb = wid // (_NX // _SLICES)
    xlo = wid * _SLICES - bb * _NX
    for z in range(_NZ):
        off = pl.multiple_of(((bb * _NZ + z) * _NX + xlo) * _NY, 8)
        pltpu.async_copy(widx.at[pl.ds(z * _TILE_W, _TILE_W)],
                         bin_hbm.at[pl.ds(off, _TILE_W)], sem1)
    for c in range(_NZ):
        off = pl.multiple_of(c * (_B * _NX * _NY) + wid * _TILE_W, 8)
        pltpu.make_async_copy(h1acc.at[pl.ds(c * _TILE_W, _TILE_W)],
                              h1_hbm.at[pl.ds(off, _TILE_W)], sem0).wait()
    for z in range(_NZ):
        off = pl.multiple_of(((bb * _NZ + z) * _NX + xlo) * _NY, 8)
        pltpu.make_async_copy(widx.at[pl.ds(z * _TILE_W, _TILE_W)],
                              bin_hbm.at[pl.ds(off, _TILE_W)], sem1).wait()


def _sc_scatter(gw, hw, p0, p1, p2, p3):
    mesh = plsc.VectorSubcoreMesh(core_axis_name="c", subcore_axis_name="s")
    fn = functools.partial(
        pl.kernel, mesh=mesh,
        compiler_params=pltpu.CompilerParams(needs_layout_passes=False),
        out_type=[
            jax.ShapeDtypeStruct((_NZ * _B * _NX * _NY,), jnp.float32),
            jax.ShapeDtypeStruct((_B * _NZ * _NX * _NY,), jnp.float32),
        ],
        scratch_types=[
            pltpu.VMEM((_CW,), jnp.float32),
            pltpu.VMEM((_CW,), jnp.float32),
            pltpu.VMEM((_CH,), jnp.int32),
            pltpu.VMEM((_CH,), jnp.int32),
            pltpu.VMEM((_CH,), jnp.int32),
            pltpu.VMEM((_CH,), jnp.int32),
            pltpu.VMEM((_CH,), jnp.float32),
            pltpu.VMEM((_CH,), jnp.float32),
            pltpu.VMEM((_CH,), jnp.float32),
            pltpu.VMEM((_CH,), jnp.float32),
            pltpu.VMEM((_CH,), jnp.float32),
            pltpu.VMEM((_CH,), jnp.float32),
            pltpu.VMEM((_CH,), jnp.float32),
            pltpu.VMEM((_CH,), jnp.float32),
            pltpu.SemaphoreType.DMA,
            pltpu.SemaphoreType.DMA,
        ],
    )(_sc_body)
    return fn(gw, hw, p0, p1, p2, p3)


def _head_body(p_ref, h1_ref, o_ref):
    a = [h1_ref[c, 0] + p_ref[0, c] for c in range(4)]
    m = (a[0] + a[1] + a[2] + a[3]) * 0.25
    d = [a[c] - m for c in range(4)]
    var = (d[0] * d[0] + d[1] * d[1] + d[2] * d[2] + d[3] * d[3]) * 0.25
    inv = lax.rsqrt(var + 1e-5)
    r = [jnp.maximum(d[c] * inv * p_ref[1, c] + p_ref[2, c], 0.0)
         for c in range(4)]
    s = [r[0] * p_ref[6, j] + r[1] * p_ref[7, j] + r[2] * p_ref[8, j]
         + r[3] * p_ref[9, j] + p_ref[3, j] for j in range(4)]
    m2 = (s[0] + s[1] + s[2] + s[3]) * 0.25
    d2 = [s[j] - m2 for j in range(4)]
    var2 = (d2[0] * d2[0] + d2[1] * d2[1] + d2[2] * d2[2]
            + d2[3] * d2[3]) * 0.25
    inv2 = lax.rsqrt(var2 + 1e-5)
    for j in range(4):
        o_ref[0, j] = d2[j] * inv2 * p_ref[4, j] + p_ref[5, j]


def _head(P, H1):
    bx = 40
    return pl.pallas_call(
        _head_body,
        grid=(_B, _NX // bx),
        in_specs=[
            pl.BlockSpec(memory_space=pltpu.SMEM),
            pl.BlockSpec((_NZ, 1, bx, _NY), lambda b, i: (0, b, i, 0)),
        ],
        out_specs=pl.BlockSpec((1, _NZ, bx, _NY), lambda b, i: (b, 0, i, 0)),
        out_shape=jax.ShapeDtypeStruct((_B, _NZ, _NX, _NY), jnp.float32),
    )(P, H1)


def kernel(voxel_features, coors, nz_embed, W1, b1, g1, be1, W2, b2, g2, be2,
           batch_size):
    p0, p1, p2, p3, gw, hw, P = _project(
        voxel_features, W1, coors.T, nz_embed, b1, g1, be1, W2, b2, g2, be2)
    h1_flat, bin_flat = _sc_scatter(gw, hw, p0, p1, p2, p3)
    H1 = h1_flat.reshape(_NZ, _B, _NX, _NY)
    out = _head(P, H1)
    binary = bin_flat.reshape(_B, _NZ, _NX, _NY)
    return out, binary


# speedup vs baseline: 1.3795x; 1.0003x over previous
import functools

import jax
import jax.numpy as jnp
from jax import lax
from jax.experimental import pallas as pl
from jax.experimental.pallas import tpu as pltpu
from jax.experimental.pallas import tpu_sc as plsc

_NX, _NY, _NZ, _C, _B, _N = 400, 400, 4, 64, 2, 30000
_NW = 32
_SLICES = (_B * _NX) // _NW
_TILE_W = _SLICES * _NY
_CW = _NZ * _TILE_W
_CH = 2000
_NCHUNK = _N // _CH
_VREGS = _CH // 16
_UNROLL = 5


def _proj_body(w1_ref, f_ref, ct_ref, ne_ref, b1_ref, g1_ref, be1_ref,
               w2_ref, b2_ref, g2_ref, be2_ref,
               p0_ref, p1_ref, p2_ref, p3_ref, gw_ref, hw_ref, prm_ref):
    p = lax.dot_general(
        w1_ref[...], f_ref[...], (((0,), (1,)), ((), ())),
        preferred_element_type=jnp.float32)
    p0_ref[...] = p[0, :]
    p1_ref[...] = p[1, :]
    p2_ref[...] = p[2, :]
    p3_ref[...] = p[3, :]
    b = jnp.minimum(ct_ref[0:1, :], _B - 1)
    z = jnp.minimum(ct_ref[1:2, :], _NZ - 1)
    x = jnp.minimum(ct_ref[2:3, :], _NX - 1)
    y = jnp.minimum(ct_ref[3:4, :], _NY - 1)
    hw = b * _NX * _NY + x * _NY + y
    hw_ref[...] = hw[0, :]
    gw_ref[...] = (z * _TILE_W + hw)[0, :]
    cvec = jnp.sum(ne_ref[...]) * jnp.sum(w1_ref[...], axis=0) + b1_ref[...]
    prm_ref[0:1, :] = cvec.reshape(1, _NZ)
    prm_ref[1:2, :] = g1_ref[...].reshape(1, _NZ)
    prm_ref[2:3, :] = be1_ref[...].reshape(1, _NZ)
    prm_ref[3:4, :] = b2_ref[...].reshape(1, _NZ)
    prm_ref[4:5, :] = g2_ref[...].reshape(1, _NZ)
    prm_ref[5:6, :] = be2_ref[...].reshape(1, _NZ)
    prm_ref[6:10, :] = w2_ref[...]


def _project(voxel_features, W1, coorsT, nz_embed, b1, g1, be1, W2, b2, g2,
             be2):
    vec_f = jax.ShapeDtypeStruct((_N,), jnp.float32)
    vec_i = jax.ShapeDtypeStruct((_N,), jnp.int32)
    return pl.pallas_call(
        _proj_body,
        out_shape=[vec_f, vec_f, vec_f, vec_f, vec_i, vec_i,
                   jax.ShapeDtypeStruct((10, _NZ), jnp.float32)],
    )(W1, voxel_features, coorsT, nz_embed, b1, g1, be1, W2, b2, g2, be2)


def _sc_body(gwh, hwh, p0h, p1h, p2h, p3h, h1_hbm, bin_hbm,
             widx, h1acc, bgw0, bgw1, bhw0, bhw1, bp00, bp01, bp10, bp11,
             bp20, bp21, bp30, bp31, sem0, sem1):
    bgw = [bgw0, bgw1]
    bhw = [bhw0, bhw1]
    bp0 = [bp00, bp01]
    bp1 = [bp10, bp11]
    bp2 = [bp20, bp21]
    bp3 = [bp30, bp31]
    wid = lax.axis_index("s") * 2 + lax.axis_index("c")
    lo4 = wid * _TILE_W
    lanef = jnp.arange(16, dtype=jnp.int32).astype(jnp.float32)

    @plsc.parallel_loop(0, _CW // 160, 1)
    def _zero(i):
        z16 = jnp.zeros((16,), jnp.float32)
        for u in range(10):
            o = pl.multiple_of((i * 10 + u) * 16, 8)
            widx[pl.ds(o, 16)] = z16
            h1acc[pl.ds(o, 16)] = z16

    def _keys(s, j):
        o = pl.multiple_of(j * 16, 8)
        hw = bhw[s][pl.ds(o, 16)]
        msk = (hw >= lo4) & (hw < lo4 + _TILE_W)
        h = jnp.clip(hw - lo4, 0, _TILE_W - 1)
        idx = jnp.clip(bgw[s][pl.ds(o, 16)] - lo4, 0, _CW - 1)
        return o, idx, h, msk

    sems = [sem0, sem1]

    def _issue1(base, s):
        base = pl.multiple_of(base, 8)
        pltpu.async_copy(gwh.at[pl.ds(base, _CH)], bgw[s], sems[s])
        pltpu.async_copy(hwh.at[pl.ds(base, _CH)], bhw[s], sems[s])

    def _wait1(s):
        pltpu.make_async_copy(gwh.at[pl.ds(0, _CH)], bgw[s], sems[s]).wait()
        pltpu.make_async_copy(hwh.at[pl.ds(0, _CH)], bhw[s], sems[s]).wait()

    def _p1_compute(s, base):
        def _p1_vec(jj, c2):
            for u in range(_UNROLL):
                j = jj * _UNROLL + u
                o, idx, h, msk = _keys(s, j)
                pid = lanef + (base + j * 16 + 1).astype(jnp.float32)
                plsc.store_scatter(widx, [idx], pid, mask=msk)
            return c2
        lax.fori_loop(0, _VREGS // _UNROLL, _p1_vec, 0)

    _issue1(0, 0)

    def _p1_pair(ci2, carry):
        base0 = ci2 * (2 * _CH)
        _wait1(0)
        _issue1(base0 + _CH, 1)
        _p1_compute(0, base0)
        _wait1(1)
        _issue1(base0 + 2 * _CH, 0)
        _p1_compute(1, base0 + _CH)
        return carry
    lax.fori_loop(0, (_NCHUNK - 1) // 2, _p1_pair, 0)
    _wait1(0)
    _p1_compute(0, (_NCHUNK - 1) * _CH)

    def _issue2(base, s):
        base = pl.multiple_of(base, 8)
        pltpu.async_copy(gwh.at[pl.ds(base, _CH)], bgw[s], sems[s])
        pltpu.async_copy(hwh.at[pl.ds(base, _CH)], bhw[s], sems[s])
        pltpu.async_copy(p0h.at[pl.ds(base, _CH)], bp0[s], sems[s])
        pltpu.async_copy(p1h.at[pl.ds(base, _CH)], bp1[s], sems[s])
        pltpu.async_copy(p2h.at[pl.ds(base, _CH)], bp2[s], sems[s])
        pltpu.async_copy(p3h.at[pl.ds(base, _CH)], bp3[s], sems[s])

    def _wait2(s):
        _wait1(s)
        pltpu.make_async_copy(p0h.at[pl.ds(0, _CH)], bp0[s], sems[s]).wait()
        pltpu.make_async_copy(p1h.at[pl.ds(0, _CH)], bp1[s], sems[s]).wait()
        pltpu.make_async_copy(p2h.at[pl.ds(0, _CH)], bp2[s], sems[s]).wait()
        pltpu.make_async_copy(p3h.at[pl.ds(0, _CH)], bp3[s], sems[s]).wait()

    def _p2_compute(s, base):
        @plsc.parallel_loop(0, _VREGS // _UNROLL, 1)
        def _p2_vec(jj):
            for u in range(_UNROLL):
                j = jj * _UNROLL + u
                o, idx, h, msk = _keys(s, j)
                pid = lanef + (base + j * 16 + 1).astype(jnp.float32)
                w = plsc.load_gather(widx, [idx], mask=msk)
                keep = msk & (w == pid)
                plsc.addupdate_scatter(h1acc, [h], bp0[s][pl.ds(o, 16)],
                                       mask=keep)
                plsc.addupdate_scatter(h1acc, [h + _TILE_W],
                                       bp1[s][pl.ds(o, 16)], mask=keep)
                plsc.addupdate_scatter(h1acc, [h + 2 * _TILE_W],
                                       bp2[s][pl.ds(o, 16)], mask=keep)
                plsc.addupdate_scatter(h1acc, [h + 3 * _TILE_W],
                                       bp3[s][pl.ds(o, 16)], mask=keep)

    _issue2(0, 0)

    def _p2_pair(ci2, carry):
        base0 = ci2 * (2 * _CH)
        _wait2(0)
        _issue2(base0 + _CH, 1)
        _p2_compute(0, base0)
        _wait2(1)
        _issue2(base0 + 2 * _CH, 0)
        _p2_compute(1, base0 + _CH)
        return carry
    lax.fori_loop(0, (_NCHUNK - 1) // 2, _p2_pair, 0)
    _wait2(0)
    _p2_compute(0, (_NCHUNK - 1) * _CH)

    for c in range(_NZ):
        off = pl.multiple_of(c * (_B * _NX * _NY) + wid * _TILE_W, 8)
        pltpu.async_copy(h1acc.at[pl.ds(c * _TILE_W, _TILE_W)],
                         h1_hbm.at[pl.ds(off, _TILE_W)], sem0)

    @plsc.parallel_loop(0, _CW // 160, 1)
    def _conv(i):
        for u in range(10):
            o = pl.multiple_of((i * 10 + u) * 16, 8)
            w = widx[pl.ds(o, 16)]
            widx[pl.ds(o, 16)] = jnp.where(w > 0.0, 1.0, 0.0)

    bb = wid // (_NX // _SLICES)
    xlo = wid * _SLICES - bb * _NX
    for z in range(_NZ):
        off = pl.multiple_of(((bb * _NZ + z) * _NX + xlo) * _NY, 8)
        pltpu.async_copy(widx.at[pl.ds(z * _TILE_W, _TILE_W)],
                         bin_hbm.at[pl.ds(off, _TILE_W)], sem1)
    for c in range(_NZ):
        off = pl.multiple_of(c * (_B * _NX * _NY) + wid * _TILE_W, 8)
        pltpu.make_async_copy(h1acc.at[pl.ds(c * _TILE_W, _TILE_W)],
                              h1_hbm.at[pl.ds(off, _TILE_W)], sem0).wait()
    for z in range(_NZ):
        off = pl.multiple_of(((bb * _NZ + z) * _NX + xlo) * _NY, 8)
        pltpu.make_async_copy(widx.at[pl.ds(z * _TILE_W, _TILE_W)],
                              bin_hbm.at[pl.ds(off, _TILE_W)], sem1).wait()


def _sc_scatter(gw, hw, p0, p1, p2, p3):
    mesh = plsc.VectorSubcoreMesh(core_axis_name="c", subcore_axis_name="s")
    fn = functools.partial(
        pl.kernel, mesh=mesh,
        compiler_params=pltpu.CompilerParams(needs_layout_passes=False),
        out_type=[
            jax.ShapeDtypeStruct((_NZ * _B * _NX * _NY,), jnp.float32),
            jax.ShapeDtypeStruct((_B * _NZ * _NX * _NY,), jnp.float32),
        ],
        scratch_types=[
            pltpu.VMEM((_CW,), jnp.float32),
            pltpu.VMEM((_CW,), jnp.float32),
            pltpu.VMEM((_CH,), jnp.int32),
            pltpu.VMEM((_CH,), jnp.int32),
            pltpu.VMEM((_CH,), jnp.int32),
            pltpu.VMEM((_CH,), jnp.int32),
            pltpu.VMEM((_CH,), jnp.float32),
            pltpu.VMEM((_CH,), jnp.float32),
            pltpu.VMEM((_CH,), jnp.float32),
            pltpu.VMEM((_CH,), jnp.float32),
            pltpu.VMEM((_CH,), jnp.float32),
            pltpu.VMEM((_CH,), jnp.float32),
            pltpu.VMEM((_CH,), jnp.float32),
            pltpu.VMEM((_CH,), jnp.float32),
            pltpu.SemaphoreType.DMA,
            pltpu.SemaphoreType.DMA,
        ],
    )(_sc_body)
    return fn(gw, hw, p0, p1, p2, p3)


def _head_body(p_ref, h1_ref, o_ref):
    a = [h1_ref[c, 0] + p_ref[0, c] for c in range(4)]
    m = (a[0] + a[1] + a[2] + a[3]) * 0.25
    d = [a[c] - m for c in range(4)]
    var = (d[0] * d[0] + d[1] * d[1] + d[2] * d[2] + d[3] * d[3]) * 0.25
    inv = lax.rsqrt(var + 1e-5)
    r = [jnp.maximum(d[c] * inv * p_ref[1, c] + p_ref[2, c], 0.0)
         for c in range(4)]
    s = [r[0] * p_ref[6, j] + r[1] * p_ref[7, j] + r[2] * p_ref[8, j]
         + r[3] * p_ref[9, j] + p_ref[3, j] for j in range(4)]
    m2 = (s[0] + s[1] + s[2] + s[3]) * 0.25
    d2 = [s[j] - m2 for j in range(4)]
    var2 = (d2[0] * d2[0] + d2[1] * d2[1] + d2[2] * d2[2]
            + d2[3] * d2[3]) * 0.25
    inv2 = lax.rsqrt(var2 + 1e-5)
    for j in range(4):
        o_ref[0, j] = d2[j] * inv2 * p_ref[4, j] + p_ref[5, j]


def _head(P, H1):
    bx = 40
    return pl.pallas_call(
        _head_body,
        grid=(_B, _NX // bx),
        in_specs=[
            pl.BlockSpec(memory_space=pltpu.SMEM),
            pl.BlockSpec((_NZ, 1, bx, _NY), lambda b, i: (0, b, i, 0)),
        ],
        out_specs=pl.BlockSpec((1, _NZ, bx, _NY), lambda b, i: (b, 0, i, 0)),
        out_shape=jax.ShapeDtypeStruct((_B, _NZ, _NX, _NY), jnp.float32),
    )(P, H1)


def kernel(voxel_features, coors, nz_embed, W1, b1, g1, be1, W2, b2, g2, be2,
           batch_size):
    p0, p1, p2, p3, gw, hw, P = _project(
        voxel_features, W1, coors.T, nz_embed, b1, g1, be1, W2, b2, g2, be2)
    h1_flat, bin_flat = _sc_scatter(gw, hw, p0, p1, p2, p3)
    H1 = h1_flat.reshape(_NZ, _B, _NX, _NY)
    out = _head(P, H1)
    binary = bin_flat.reshape(_B, _NZ, _NX, _NY)
    return out, binary
